# Initial kernel scaffold; baseline (speedup 1.0000x reference)
import jax
import jax.numpy as jnp
from jax.experimental import pallas as pl


def kernel(coords, features, ec1_W1, ec1_b1, ec1_W2, ec1_b2, ec2_W1, ec2_b1, ec2_W2, ec2_b2, ec3_W1, ec3_b1, ec3_W2, ec3_b2, out_W1, out_b1, out_W2, out_b2, out_W3, out_b3):
    raise NotImplementedError("write your pallas kernel here")


# speedup vs baseline: 6.6282x; 6.6282x over previous
import jax
import jax.numpy as jnp
from jax.experimental import pallas as pl
from jax.experimental.pallas import tpu as pltpu

_B, _P, _K = 1024, 100, 4
_PP = 104
_J = 8
_HI = jax.lax.Precision.HIGHEST


def _mm(a, b):
    return jax.lax.dot_general(a, b, (((1,), (0,)), ((), ())),
                               precision=_HI,
                               preferred_element_type=jnp.float32)


def _mm_t(a, b):
    return jax.lax.dot_general(a, b, (((1,), (1,)), ((), ())),
                               precision=_HI,
                               preferred_element_type=jnp.float32)


def _elu(x):
    return jnp.where(x > 0, x, jnp.exp(jnp.minimum(x, 0.0)) - 1.0)


def _mask_d2(d2):
    lane = jax.lax.broadcasted_iota(jnp.int32, (_PP, _PP), 1)
    row = jax.lax.broadcasted_iota(jnp.int32, (_PP, _PP), 0)
    return jnp.where((lane == row) | (lane >= _P), 1e9, d2)


def _pairwise_d2(x):
    s = jnp.sum(x * x, axis=1, keepdims=True)
    return _mask_d2(s + s.T - 2.0 * _mm_t(x, x))


def _pairwise_d2_coords(c):
    a = c[:, 0:1]
    b = c[:, 1:2]
    return _mask_d2((a - a.T) ** 2 + (b - b.T) ** 2)


def _topk_onehots(D):
    lane = jax.lax.broadcasted_iota(jnp.int32, D.shape, 1)
    ohs = []
    for _ in range(_K):
        m = jnp.min(D, axis=1, keepdims=True)
        idx = jnp.min(jnp.where(D <= m, lane, _PP), axis=1, keepdims=True)
        oh = lane == idx
        ohs.append(oh.astype(jnp.float32))
        D = jnp.where(oh, 1e9, D)
    return ohs


def _edge_conv(X, ohs, W1d, W1b, b1, W2, b2):
    Ap = _mm(X, W1d)
    Bm = _mm(X, W1b)
    es = []
    for j in range(_J):
        sl = slice(j * _PP, (j + 1) * _PP)
        S = jnp.concatenate([ohs[s][sl] for s in range(_K)], axis=0)
        g = _mm(S, Bm[sl])
        es.append(g + jnp.concatenate([Ap[sl]] * _K, axis=0))
    E = jnp.concatenate(es, axis=0) + b1
    H = _elu(E)
    H = _elu(_mm(H, W2) + b2)
    outs = []
    for j in range(_J):
        h4 = H[j * _K * _PP:(j + 1) * _K * _PP]
        outs.append(h4[0:_PP] + h4[_PP:2 * _PP]
                    + h4[2 * _PP:3 * _PP] + h4[3 * _PP:4 * _PP])
    return 0.25 * jnp.concatenate(outs, axis=0)


def _net_kernel(coords_ref, feats_ref,
                e1d_ref, e1b_ref, e1b1_ref, e1W2_ref, e1b2_ref,
                e2d_ref, e2b_ref, e2b1_ref, e2W2_ref, e2b2_ref,
                e3d_ref, e3b_ref, e3b1_ref, e3W2_ref, e3b2_ref,
                v1_ref, v2_ref, v3_ref, v4_ref,
                ob1_ref, oW2_ref, ob2_ref, oW3_ref, ob3_ref,
                out_ref):
    C = coords_ref[...]
    F = feats_ref[...]

    D1 = jnp.concatenate(
        [_pairwise_d2_coords(C[j * _PP:(j + 1) * _PP]) for j in range(_J)],
        axis=0)
    out1 = _edge_conv(F, _topk_onehots(D1), e1d_ref[...], e1b_ref[...],
                      e1b1_ref[...], e1W2_ref[...], e1b2_ref[...])

    D2 = jnp.concatenate(
        [_pairwise_d2(out1[j * _PP:(j + 1) * _PP]) for j in range(_J)],
        axis=0)
    out2 = _edge_conv(out1, _topk_onehots(D2), e2d_ref[...], e2b_ref[...],
                      e2b1_ref[...], e2W2_ref[...], e2b2_ref[...])

    D3 = jnp.concatenate(
        [_pairwise_d2(out2[j * _PP:(j + 1) * _PP]) for j in range(_J)],
        axis=0)
    out3 = _edge_conv(out2, _topk_onehots(D3), e3d_ref[...], e3b_ref[...],
                      e3b1_ref[...], e3W2_ref[...], e3b2_ref[...])

    h = _elu(_mm(F, v1_ref[...]) + _mm(out1, v2_ref[...])
                   + _mm(out2, v3_ref[...]) + _mm(out3, v4_ref[...])
                   + ob1_ref[...])
    h = _elu(_mm(h, oW2_ref[...]) + ob2_ref[...])
    out_ref[...] = _mm(h, oW3_ref[...]) + ob3_ref[...]


def kernel(coords, features,
           ec1_W1, ec1_b1, ec1_W2, ec1_b2,
           ec2_W1, ec2_b1, ec2_W2, ec2_b2,
           ec3_W1, ec3_b1, ec3_W2, ec3_b2,
           out_W1, out_b1, out_W2, out_b2, out_W3, out_b3):
    pad = ((0, 0), (0, _PP - _P), (0, 0))
    c2 = jnp.pad(coords, pad).reshape(_B * _PP, 2)
    f2 = jnp.pad(features, pad).reshape(_B * _PP, 5)

    def split_w1(W1, d):
        return W1[:d] - W1[d:], W1[d:]

    e1d, e1b = split_w1(ec1_W1, 5)
    e2d, e2b = split_w1(ec2_W1, 64)
    e3d, e3b = split_w1(ec3_W1, 128)
    v1, v2, v3, v4 = out_W1[:5], out_W1[5:69], out_W1[69:197], out_W1[197:]
    row = lambda b: b.reshape(1, -1)

    args = (c2, f2,
            e1d, e1b, row(ec1_b1), ec1_W2, row(ec1_b2),
            e2d, e2b, row(ec2_b1), ec2_W2, row(ec2_b2),
            e3d, e3b, row(ec3_b1), ec3_W2, row(ec3_b2),
            v1, v2, v3, v4,
            row(out_b1), out_W2, row(out_b2), out_W3, row(out_b3))

    blk = _J * _PP
    in_specs = [pl.BlockSpec((blk, 2), lambda i: (i, 0)),
                pl.BlockSpec((blk, 5), lambda i: (i, 0))]
    for a in args[2:]:
        in_specs.append(pl.BlockSpec(a.shape, lambda i: (0, 0)))

    out = pl.pallas_call(
        _net_kernel,
        grid=(_B // _J,),
        in_specs=in_specs,
        out_specs=pl.BlockSpec((blk, 2), lambda i: (i, 0)),
        out_shape=jax.ShapeDtypeStruct((_B * _PP, 2), jnp.float32),
        compiler_params=pltpu.CompilerParams(
            dimension_semantics=("parallel",)),
    )(*args)
    return out.reshape(_B, _PP, 2)[:, :_P, :]


# speedup vs baseline: 11.2616x; 1.6991x over previous
import jax
import jax.numpy as jnp
from jax.experimental import pallas as pl
from jax.experimental.pallas import tpu as pltpu

_B, _P, _K = 1024, 100, 4
_PP = 104
_J = 8
_BIG = 0x7F000000


def _mm_p(a, b, prec):
    return jax.lax.dot_general(a, b, (((1,), (0,)), ((), ())),
                               precision=prec,
                               preferred_element_type=jnp.float32)


def _mm3(a, b):
    return _mm_p(a, b, jax.lax.Precision.HIGHEST)


def _mm1(a, b):
    return _mm_p(a, b, jax.lax.Precision.DEFAULT)


def _mmt3(a, b):
    return jax.lax.dot_general(a, b, (((1,), (1,)), ((), ())),
                               precision=jax.lax.Precision.HIGHEST,
                               preferred_element_type=jnp.float32)


def _elu(x):
    return jnp.where(x > 0, x, jnp.exp(jnp.minimum(x, 0.0)) - 1.0)


def _pack_d2(d2):
    lane = jax.lax.broadcasted_iota(jnp.int32, (_PP, _PP), 1)
    row = jax.lax.broadcasted_iota(jnp.int32, (_PP, _PP), 0)
    di = jax.lax.bitcast_convert_type(jnp.maximum(d2, 0.0), jnp.int32)
    di = jax.lax.bitwise_or(jax.lax.bitwise_and(di + 64, jnp.int32(~127)), lane)
    return jnp.where((lane == row) | (lane >= _P), _BIG + lane, di)


def _pairwise_d2(x):
    s = jnp.sum(x * x, axis=1, keepdims=True)
    return _pack_d2(s + s.T - 2.0 * _mmt3(x, x))


def _pairwise_d2_coords(c):
    a = c[:, 0:1]
    b = c[:, 1:2]
    return _pack_d2((a - a.T) ** 2 + (b - b.T) ** 2)


def _topk_onehots(D):
    ohs = []
    for _ in range(_K):
        m = jnp.min(D, axis=1, keepdims=True)
        oh = D == m
        ohs.append(oh.astype(jnp.float32))
        D = jnp.where(oh, _BIG, D)
    return ohs


def _edge_conv(X, ohs, W1d, W1b, b1, W2, b2, mm):
    Ap = mm(X, W1d)
    Bm = mm(X, W1b)
    es = []
    for j in range(_J):
        sl = slice(j * _PP, (j + 1) * _PP)
        S = jnp.concatenate([ohs[s][sl] for s in range(_K)], axis=0)
        g = mm(S, Bm[sl])
        es.append(g + jnp.concatenate([Ap[sl]] * _K, axis=0))
    E = jnp.concatenate(es, axis=0) + b1
    H = _elu(E)
    H = _elu(mm(H, W2) + b2)
    outs = []
    for j in range(_J):
        h4 = H[j * _K * _PP:(j + 1) * _K * _PP]
        outs.append(h4[0:_PP] + h4[_PP:2 * _PP]
                    + h4[2 * _PP:3 * _PP] + h4[3 * _PP:4 * _PP])
    return 0.25 * jnp.concatenate(outs, axis=0)


def _net_kernel(coords_ref, feats_ref,
                e1d_ref, e1b_ref, e1b1_ref, e1W2_ref, e1b2_ref,
                e2d_ref, e2b_ref, e2b1_ref, e2W2_ref, e2b2_ref,
                e3d_ref, e3b_ref, e3b1_ref, e3W2_ref, e3b2_ref,
                v1_ref, v2_ref, v3_ref, v4_ref,
                ob1_ref, oW2_ref, ob2_ref, oW3_ref, ob3_ref,
                out_ref):
    C = coords_ref[...]
    F = feats_ref[...]

    D1 = jnp.concatenate(
        [_pairwise_d2_coords(C[j * _PP:(j + 1) * _PP]) for j in range(_J)],
        axis=0)
    out1 = _edge_conv(F, _topk_onehots(D1), e1d_ref[...], e1b_ref[...],
                      e1b1_ref[...], e1W2_ref[...], e1b2_ref[...], _mm3)

    D2 = jnp.concatenate(
        [_pairwise_d2(out1[j * _PP:(j + 1) * _PP]) for j in range(_J)],
        axis=0)
    out2 = _edge_conv(out1, _topk_onehots(D2), e2d_ref[...], e2b_ref[...],
                      e2b1_ref[...], e2W2_ref[...], e2b2_ref[...], _mm3)

    D3 = jnp.concatenate(
        [_pairwise_d2(out2[j * _PP:(j + 1) * _PP]) for j in range(_J)],
        axis=0)
    out3 = _edge_conv(out2, _topk_onehots(D3), e3d_ref[...], e3b_ref[...],
                      e3b1_ref[...], e3W2_ref[...], e3b2_ref[...], _mm1)

    h = _elu(_mm1(F, v1_ref[...]) + _mm1(out1, v2_ref[...])
             + _mm1(out2, v3_ref[...]) + _mm1(out3, v4_ref[...])
             + ob1_ref[...])
    h = _elu(_mm1(h, oW2_ref[...]) + ob2_ref[...])
    out_ref[...] = _mm1(h, oW3_ref[...]) + ob3_ref[...]


def kernel(coords, features,
           ec1_W1, ec1_b1, ec1_W2, ec1_b2,
           ec2_W1, ec2_b1, ec2_W2, ec2_b2,
           ec3_W1, ec3_b1, ec3_W2, ec3_b2,
           out_W1, out_b1, out_W2, out_b2, out_W3, out_b3):
    pad = ((0, 0), (0, _PP - _P), (0, 0))
    c2 = jnp.pad(coords, pad).reshape(_B * _PP, 2)
    f2 = jnp.pad(features, pad).reshape(_B * _PP, 5)

    def split_w1(W1, d):
        return W1[:d] - W1[d:], W1[d:]

    e1d, e1b = split_w1(ec1_W1, 5)
    e2d, e2b = split_w1(ec2_W1, 64)
    e3d, e3b = split_w1(ec3_W1, 128)
    v1, v2, v3, v4 = out_W1[:5], out_W1[5:69], out_W1[69:197], out_W1[197:]
    row = lambda b: b.reshape(1, -1)

    args = (c2, f2,
            e1d, e1b, row(ec1_b1), ec1_W2, row(ec1_b2),
            e2d, e2b, row(ec2_b1), ec2_W2, row(ec2_b2),
            e3d, e3b, row(ec3_b1), ec3_W2, row(ec3_b2),
            v1, v2, v3, v4,
            row(out_b1), out_W2, row(out_b2), out_W3, row(out_b3))

    blk = _J * _PP
    in_specs = [pl.BlockSpec((blk, 2), lambda i: (i, 0)),
                pl.BlockSpec((blk, 5), lambda i: (i, 0))]
    for a in args[2:]:
        in_specs.append(pl.BlockSpec(a.shape, lambda i: (0, 0)))

    out = pl.pallas_call(
        _net_kernel,
        grid=(_B // _J,),
        in_specs=in_specs,
        out_specs=pl.BlockSpec((blk, 2), lambda i: (i, 0)),
        out_shape=jax.ShapeDtypeStruct((_B * _PP, 2), jnp.float32),
        compiler_params=pltpu.CompilerParams(
            dimension_semantics=("parallel",)),
    )(*args)
    return out.reshape(_B, _PP, 2)[:, :_P, :]


# speedup vs baseline: 12.4650x; 1.1069x over previous
import jax
import jax.numpy as jnp
from jax.experimental import pallas as pl
from jax.experimental.pallas import tpu as pltpu

_B, _P, _K = 1024, 100, 4
_PP = 104
_J = 8
_LW = 128
_BIG = 0x7F000000


def _mm_p(a, b, prec):
    return jax.lax.dot_general(a, b, (((1,), (0,)), ((), ())),
                               precision=prec,
                               preferred_element_type=jnp.float32)


def _mm3(a, b):
    return _mm_p(a, b, jax.lax.Precision.HIGHEST)


def _mm1(a, b):
    return _mm_p(a, b, None)


def _mmt(a, b):
    return jax.lax.dot_general(a, b, (((1,), (1,)), ((), ())),
                               precision=jax.lax.Precision.HIGHEST,
                               preferred_element_type=jnp.float32)


def _elu(x):
    return jnp.where(x > 0, x, jnp.exp(x) - 1.0)


def _pairwise_d2(x):
    s = jnp.sum(x * x, axis=1, keepdims=True)
    return s + s.T - 2.0 * _mmt(x, x)


def _pairwise_d2_coords(c):
    a = c[:, 0:1]
    b = c[:, 1:2]
    return (a - a.T) ** 2 + (b - b.T) ** 2


def _pack(d2, cand_axis):
    cand = jax.lax.broadcasted_iota(jnp.int32, (_PP, _PP), cand_axis)
    other = jax.lax.broadcasted_iota(jnp.int32, (_PP, _PP), 1 - cand_axis)
    di = jax.lax.bitcast_convert_type(jnp.maximum(d2, 0.0), jnp.int32)
    di = jax.lax.bitwise_or(jax.lax.bitwise_and(di + 64, jnp.int32(~127)),
                            cand)
    return jnp.where((cand == other) | (cand >= _P), _BIG + cand, di)


def _min4(d2s):
    DT = jnp.concatenate(
        [jnp.pad(_pack(d2, 0), ((0, 0), (0, _LW - _PP)),
                 constant_values=_BIG) for d2 in d2s], axis=1)
    ms = []
    for s in range(_K):
        m = jnp.min(DT, axis=0, keepdims=True)
        ms.append(m)
        if s + 1 < _K:
            DT = jnp.where(DT == m, _BIG, DT)
    return jnp.concatenate(ms, axis=0).T


def _edge_conv(X, d2s, msT, W1d, W1b, b1, W2, b2, mm):
    Ap = mm(X, W1d)
    Bm = mm(X, W1b)
    es = []
    for j in range(_J):
        sl = slice(j * _PP, (j + 1) * _PP)
        DL = _pack(d2s[j], 1)
        mj = msT[j * _LW:j * _LW + _PP, :]
        S = jnp.concatenate(
            [(DL == mj[:, s:s + 1]).astype(jnp.float32) for s in range(_K)],
            axis=0)
        g = mm(S, Bm[sl])
        es.append(g + jnp.concatenate([Ap[sl]] * _K, axis=0))
    E = jnp.concatenate(es, axis=0) + b1
    H = _elu(E)
    H = _elu(mm(H, W2) + b2)
    outs = []
    for j in range(_J):
        h4 = H[j * _K * _PP:(j + 1) * _K * _PP]
        outs.append(h4[0:_PP] + h4[_PP:2 * _PP]
                    + h4[2 * _PP:3 * _PP] + h4[3 * _PP:4 * _PP])
    return 0.25 * jnp.concatenate(outs, axis=0)


def _net_kernel(coords_ref, feats_ref,
                e1d_ref, e1b_ref, e1b1_ref, e1W2_ref, e1b2_ref,
                e2d_ref, e2b_ref, e2b1_ref, e2W2_ref, e2b2_ref,
                e3d_ref, e3b_ref, e3b1_ref, e3W2_ref, e3b2_ref,
                v1_ref, v2_ref, v3_ref, v4_ref,
                ob1_ref, oW2_ref, ob2_ref, oW3_ref, ob3_ref,
                out_ref):
    C = coords_ref[...]
    F = feats_ref[...]

    def sel(X, pairwise):
        d2s = [pairwise(X[j * _PP:(j + 1) * _PP]) for j in range(_J)]
        return d2s, _min4(d2s)

    d2s, msT = sel(C, _pairwise_d2_coords)
    out1 = _edge_conv(F, d2s, msT, e1d_ref[...], e1b_ref[...],
                      e1b1_ref[...], e1W2_ref[...], e1b2_ref[...], _mm3)
    d2s, msT = sel(out1, _pairwise_d2)
    out2 = _edge_conv(out1, d2s, msT, e2d_ref[...], e2b_ref[...],
                      e2b1_ref[...], e2W2_ref[...], e2b2_ref[...], _mm3)
    d2s, msT = sel(out2, _pairwise_d2)
    out3 = _edge_conv(out2, d2s, msT, e3d_ref[...], e3b_ref[...],
                      e3b1_ref[...], e3W2_ref[...], e3b2_ref[...], _mm1)

    h = _elu(_mm1(F, v1_ref[...]) + _mm1(out1, v2_ref[...])
             + _mm1(out2, v3_ref[...]) + _mm1(out3, v4_ref[...])
             + ob1_ref[...])
    h = _elu(_mm1(h, oW2_ref[...]) + ob2_ref[...])
    out_ref[...] = _mm1(h, oW3_ref[...]) + ob3_ref[...]


def kernel(coords, features,
           ec1_W1, ec1_b1, ec1_W2, ec1_b2,
           ec2_W1, ec2_b1, ec2_W2, ec2_b2,
           ec3_W1, ec3_b1, ec3_W2, ec3_b2,
           out_W1, out_b1, out_W2, out_b2, out_W3, out_b3):
    pad = ((0, 0), (0, _PP - _P), (0, 0))
    c2 = jnp.pad(coords, pad).reshape(_B * _PP, 2)
    f2 = jnp.pad(features, pad).reshape(_B * _PP, 5)

    def split_w1(W1, d):
        return W1[:d] - W1[d:], W1[d:]

    e1d, e1b = split_w1(ec1_W1, 5)
    e2d, e2b = split_w1(ec2_W1, 64)
    e3d, e3b = split_w1(ec3_W1, 128)
    v1, v2, v3, v4 = out_W1[:5], out_W1[5:69], out_W1[69:197], out_W1[197:]
    row = lambda b: b.reshape(1, -1)

    args = (c2, f2,
            e1d, e1b, row(ec1_b1), ec1_W2, row(ec1_b2),
            e2d, e2b, row(ec2_b1), ec2_W2, row(ec2_b2),
            e3d, e3b, row(ec3_b1), ec3_W2, row(ec3_b2),
            v1, v2, v3, v4,
            row(out_b1), out_W2, row(out_b2), out_W3, row(out_b3))

    blk = _J * _PP
    in_specs = [pl.BlockSpec((blk, 2), lambda i: (i, 0)),
                pl.BlockSpec((blk, 5), lambda i: (i, 0))]
    for a in args[2:]:
        in_specs.append(pl.BlockSpec(a.shape, lambda i: (0, 0)))

    out = pl.pallas_call(
        _net_kernel,
        grid=(_B // _J,),
        in_specs=in_specs,
        out_specs=pl.BlockSpec((blk, 2), lambda i: (i, 0)),
        out_shape=jax.ShapeDtypeStruct((_B * _PP, 2), jnp.float32),
        compiler_params=pltpu.CompilerParams(
            dimension_semantics=("parallel",)),
    )(*args)
    return out.reshape(_B, _PP, 2)[:, :_P, :]
